# baseline (speedup 1.0000x reference)
import functools

import jax
import jax.numpy as jnp
import numpy as np
from jax import lax
from jax.experimental import pallas as pl
from jax.experimental.pallas import tpu as pltpu
from jax.experimental.pallas import tpu_sc as plsc

N = 10000
E = 320000
H = 128
HH = H // 2
NGAUSS = 10
NGRAPH = 64
CUTOFF = 10.0

NC = 2
NS = 16
LANES = 16

_MESH = plsc.VectorSubcoreMesh(
    core_axis_name="c", subcore_axis_name="s", num_cores=NC, num_subcores=NS
)
_SC_PARAMS = pltpu.CompilerParams(needs_layout_passes=False,
                                  use_tc_tiling_on_sc=False)

_STEP = np.float32(CUTOFF / (NGAUSS - 1))
_COEFF = np.float32(-0.5) / _STEP**2
_LOG2 = np.float32(np.log(2.0))


def _ssp(x):
    return jnp.maximum(x, 0.0) + jnp.log1p(jnp.exp(-jnp.abs(x))) - _LOG2


EPW = E // (NC * NS)


def _dist2_body(px_h, py_h, pz_h, src_h, dst_h, out_h, px, py, pz, sv, dv, ov):
    c = lax.axis_index("c")
    s = lax.axis_index("s")
    wid = s * NC + c
    base = wid * EPW
    pltpu.sync_copy(px_h, px)
    pltpu.sync_copy(py_h, py)
    pltpu.sync_copy(pz_h, pz)
    pltpu.sync_copy(src_h.at[pl.ds(base, EPW)], sv)
    pltpu.sync_copy(dst_h.at[pl.ds(base, EPW)], dv)

    def body(i, carry):
        si = sv[pl.ds(i * LANES, LANES)]
        di = dv[pl.ds(i * LANES, LANES)]
        dx = plsc.load_gather(px, [si]) - plsc.load_gather(px, [di])
        dy = plsc.load_gather(py, [si]) - plsc.load_gather(py, [di])
        dz = plsc.load_gather(pz, [si]) - plsc.load_gather(pz, [di])
        ov[pl.ds(i * LANES, LANES)] = dx * dx + dy * dy + dz * dz
        return carry

    lax.fori_loop(0, EPW // LANES, body, 0)
    pltpu.sync_copy(ov, out_h.at[pl.ds(base, EPW)])


_dist2 = pl.kernel(
    _dist2_body,
    out_type=jax.ShapeDtypeStruct((E,), jnp.float32),
    mesh=_MESH,
    compiler_params=_SC_PARAMS,
    scratch_types=[
        pltpu.VMEM((N,), jnp.float32),
        pltpu.VMEM((N,), jnp.float32),
        pltpu.VMEM((N,), jnp.float32),
        pltpu.VMEM((EPW,), jnp.int32),
        pltpu.VMEM((EPW,), jnp.int32),
        pltpu.VMEM((EPW,), jnp.float32),
    ],
)


K = 80
EPT = E // NS
NCHUNK = EPT // K
RPT = N // NS
ZR = 125


def _agg_body(xl_h, wf_h, src_h, dst_h, agg_h,
              xl_s, agg_s, sv, dv, gv, wv, zv,
              sem_is, sem_id, sem_g, sem_w, sem_s):
    c = lax.axis_index("c")
    s = lax.axis_index("s")
    row0 = s * RPT
    base = s * EPT
    cols = pl.ds(c * HH, HH)

    def zbody(i, carry):
        zero = jnp.zeros((LANES,), jnp.float32)
        for j in range(HH // LANES):
            zv[i, pl.ds(j * LANES, LANES)] = zero
        return carry

    lax.fori_loop(0, ZR, zbody, 0)
    for r in range(RPT // ZR):
        pltpu.sync_copy(zv, agg_s.at[pl.ds(row0 + r * ZR, ZR), :])

    pltpu.sync_copy(xl_h.at[pl.ds(row0, RPT), cols],
                    xl_s.at[pl.ds(row0, RPT), :])
    plsc.subcore_barrier()

    def issue_idx(t, b4):
        e0 = base + t * K
        pltpu.async_copy(src_h.at[pl.ds(e0, K)], sv.at[t % 2], sem_is.at[t % 2])
        pltpu.async_copy(dst_h.at[pl.ds(e0, K)], dv.at[b4], sem_id.at[b4])

    def wait_idx(t, b4):
        e0 = base + t * K
        pltpu.make_async_copy(src_h.at[pl.ds(e0, K)], sv.at[t % 2],
                              sem_is.at[t % 2]).wait()
        pltpu.make_async_copy(dst_h.at[pl.ds(e0, K)], dv.at[b4],
                              sem_id.at[b4]).wait()

    def issue_fetch(t, b):
        e0 = base + t * K
        pltpu.async_copy(xl_s.at[sv.at[b]], gv.at[b], sem_g.at[b])
        pltpu.async_copy(wf_h.at[pl.ds(e0, K), cols], wv.at[b], sem_w.at[b])

    def wait_fetch(t, b):
        e0 = base + t * K
        pltpu.make_async_copy(xl_s.at[sv.at[b]], gv.at[b], sem_g.at[b]).wait()
        pltpu.make_async_copy(wf_h.at[pl.ds(e0, K), cols], wv.at[b],
                              sem_w.at[b]).wait()

    def mul(b):
        def body(i, c2):
            for j2 in range(HH // LANES):
                sl = pl.ds(j2 * LANES, LANES)
                gv[b, i, sl] = gv[b, i, sl] * wv[b, i, sl]
            return c2

        lax.fori_loop(0, K, body, 0, unroll=4)

    def issue_scatter(b, b4):
        pltpu.async_copy(gv.at[b], agg_s.at[dv.at[b4]], sem_s.at[b], add=True)

    def wait_scatter(b, b4):
        pltpu.make_async_copy(gv.at[b], agg_s.at[dv.at[b4]],
                              sem_s.at[b]).wait()

    def substep(t, first=False, nm1=False, last=False):
        b, b4 = t % 2, t % 4
        wait_fetch(t, b)
        mul(b)
        issue_scatter(b, b4)
        if not last:
            wait_idx(t + 1, (t + 1) % 4)
        if not first:
            wait_scatter(1 - b, (t - 1) % 4)
        if not last:
            issue_fetch(t + 1, 1 - b)
        if not (nm1 or last):
            issue_idx(t + 2, (t + 2) % 4)

    issue_idx(0, 0)
    issue_idx(1, 1)
    wait_idx(0, 0)
    issue_fetch(0, 0)
    substep(0, first=True)
    substep(1)

    def quad(v, carry):
        t0 = 2 + 4 * v
        for q in range(4):
            substep(t0 + q)
        return carry

    lax.fori_loop(0, (NCHUNK - 6) // 4, quad, 0)
    substep(NCHUNK - 4)
    substep(NCHUNK - 3)
    substep(NCHUNK - 2, nm1=True)
    substep(NCHUNK - 1, last=True)
    wait_scatter((NCHUNK - 1) % 2, (NCHUNK - 1) % 4)

    plsc.subcore_barrier()
    pltpu.sync_copy(agg_s.at[pl.ds(row0, RPT), :],
                    agg_h.at[pl.ds(row0, RPT), cols])


_agg = pl.kernel(
    _agg_body,
    out_type=jax.ShapeDtypeStruct((N, H), jnp.float32),
    mesh=_MESH,
    compiler_params=_SC_PARAMS,
    scratch_types=[
        pltpu.VMEM_SHARED((N, HH), jnp.float32),
        pltpu.VMEM_SHARED((N, HH), jnp.float32),
        pltpu.VMEM((2, K), jnp.int32),
        pltpu.VMEM((4, K), jnp.int32),
        pltpu.VMEM((2, K, HH), jnp.float32),
        pltpu.VMEM((2, K, HH), jnp.float32),
        pltpu.VMEM((ZR, HH), jnp.float32),
        pltpu.SemaphoreType.DMA((2,)),
        pltpu.SemaphoreType.DMA((4,)),
        pltpu.SemaphoreType.DMA((2,)),
        pltpu.SemaphoreType.DMA((2,)),
        pltpu.SemaphoreType.DMA((2,)),
    ],
)


EP = 327680
TE = 2048
TB = TE // 128
GE = EP // TE
ER = EP // 128
TN = 2000
GN = N // TN


def _filter_body(d2_ref, mw1_ref, mb1_ref, mw2_ref, mb2_ref, wf_ref):
    d2d = d2_ref[...]
    distd = jnp.sqrt(d2d + 1e-12)
    cenvd = 0.5 * (jnp.cos(distd * jnp.pi / CUTOFF) + 1.0)
    offc = (lax.broadcasted_iota(jnp.int32, (NGAUSS, 128), 0)
            .astype(jnp.float32) * _STEP)
    mw1 = mw1_ref[...]
    rows = []
    for r in range(TB):
        distb = jnp.broadcast_to(distd[r:r + 1, :], (NGAUSS, 128))
        rbf_r = jnp.exp(_COEFF * (distb - offc) ** 2)
        t_r = lax.dot_general(rbf_r, mw1, (((0,), (0,)), ((), ())),
                              preferred_element_type=jnp.float32)
        rows.append(t_r[None])
    t = jnp.concatenate(rows, axis=0)
    cenv3 = jnp.broadcast_to(cenvd[:, :, None], (TB, 128, H))
    s = _ssp(t + mb1_ref[...].reshape(1, 1, H)) * cenv3
    wf = jnp.dot(s.reshape(TE, H), mw2_ref[...],
                 preferred_element_type=jnp.float32)
    wf = wf + (cenv3 * mb2_ref[...].reshape(1, 1, H)).reshape(TE, H)
    wf_ref[...] = wf


_filter = pl.pallas_call(
    _filter_body,
    grid=(GE,),
    in_specs=[
        pl.BlockSpec((TB, 128), lambda i: (i, 0)),
        pl.BlockSpec((NGAUSS, H), lambda i: (0, 0)),
        pl.BlockSpec((1, H), lambda i: (0, 0)),
        pl.BlockSpec((H, H), lambda i: (0, 0)),
        pl.BlockSpec((1, H), lambda i: (0, 0)),
    ],
    out_specs=pl.BlockSpec((TE, H), lambda i: (i, 0)),
    out_shape=jax.ShapeDtypeStruct((EP, H), jnp.float32),
)


def _embed_body(z_ref, emb_ref, l1w_ref, h_ref, xl_ref):
    z = z_ref[...]
    oh = (z == lax.broadcasted_iota(jnp.int32, (TN, 100), 1)).astype(jnp.float32)
    h = jnp.dot(oh, emb_ref[...], preferred_element_type=jnp.float32)
    xl = jnp.dot(h, l1w_ref[...], preferred_element_type=jnp.float32)
    h_ref[...] = h
    xl_ref[...] = xl


_embed = pl.pallas_call(
    _embed_body,
    grid=(GN,),
    in_specs=[
        pl.BlockSpec((TN, 1), lambda i: (i, 0)),
        pl.BlockSpec((100, H), lambda i: (0, 0)),
        pl.BlockSpec((H, H), lambda i: (0, 0)),
    ],
    out_specs=[
        pl.BlockSpec((TN, H), lambda i: (i, 0)),
        pl.BlockSpec((TN, H), lambda i: (i, 0)),
    ],
    out_shape=[
        jax.ShapeDtypeStruct((N, H), jnp.float32),
        jax.ShapeDtypeStruct((N, H), jnp.float32),
    ],
)


def _update_body(agg_ref, h_ref, l2w_ref, l2b_ref, lw_ref, lb_ref,
                 l1wn_ref, hn_ref, xl_ref):
    x = jnp.dot(agg_ref[...], l2w_ref[...], preferred_element_type=jnp.float32)
    x = _ssp(x + l2b_ref[...])
    x = jnp.dot(x, lw_ref[...], preferred_element_type=jnp.float32) + lb_ref[...]
    hn = h_ref[...] + x
    xl = jnp.dot(hn, l1wn_ref[...], preferred_element_type=jnp.float32)
    hn_ref[...] = hn
    xl_ref[...] = xl


_update = pl.pallas_call(
    _update_body,
    grid=(GN,),
    in_specs=[
        pl.BlockSpec((TN, H), lambda i: (i, 0)),
        pl.BlockSpec((TN, H), lambda i: (i, 0)),
        pl.BlockSpec((H, H), lambda i: (0, 0)),
        pl.BlockSpec((1, H), lambda i: (0, 0)),
        pl.BlockSpec((H, H), lambda i: (0, 0)),
        pl.BlockSpec((1, H), lambda i: (0, 0)),
        pl.BlockSpec((H, H), lambda i: (0, 0)),
    ],
    out_specs=[
        pl.BlockSpec((TN, H), lambda i: (i, 0)),
        pl.BlockSpec((TN, H), lambda i: (i, 0)),
    ],
    out_shape=[
        jax.ShapeDtypeStruct((N, H), jnp.float32),
        jax.ShapeDtypeStruct((N, H), jnp.float32),
    ],
)


def _final_body(agg_ref, h_ref, batch_ref, l2w_ref, l2b_ref, lw_ref,
                lb_ref, fl1w_ref, fl1b_ref, fl2w_ref, fl2b_ref, pw_ref, pb_ref,
                out_ref):
    i = pl.program_id(0)
    x = jnp.dot(agg_ref[...], l2w_ref[...], preferred_element_type=jnp.float32)
    x = _ssp(x + l2b_ref[...])
    x = jnp.dot(x, lw_ref[...], preferred_element_type=jnp.float32) + lb_ref[...]
    h2 = h_ref[...] + x
    hf = _ssp(jnp.dot(h2, fl1w_ref[...], preferred_element_type=jnp.float32)
              + fl1b_ref[...])
    hf = jnp.dot(hf, fl2w_ref[...], preferred_element_type=jnp.float32)
    hf = hf + fl2b_ref[...]
    hp = jnp.dot(hf, pw_ref[...], preferred_element_type=jnp.float32)
    oh = (batch_ref[...] == lax.broadcasted_iota(jnp.int32, (TN, NGRAPH), 1))
    part = lax.dot_general(oh.astype(jnp.float32), hp,
                           (((0,), (0,)), ((), ())),
                           preferred_element_type=jnp.float32)

    @pl.when(i == 0)
    def _():
        out_ref[...] = part + pb_ref[...]

    @pl.when(i > 0)
    def _():
        out_ref[...] = out_ref[...] + part


_final = pl.pallas_call(
    _final_body,
    grid=(GN,),
    in_specs=[
        pl.BlockSpec((TN, H), lambda i: (i, 0)),
        pl.BlockSpec((TN, H), lambda i: (i, 0)),
        pl.BlockSpec((TN, 1), lambda i: (i, 0)),
        pl.BlockSpec((H, H), lambda i: (0, 0)),
        pl.BlockSpec((1, H), lambda i: (0, 0)),
        pl.BlockSpec((H, H), lambda i: (0, 0)),
        pl.BlockSpec((1, H), lambda i: (0, 0)),
        pl.BlockSpec((H, HH), lambda i: (0, 0)),
        pl.BlockSpec((1, HH), lambda i: (0, 0)),
        pl.BlockSpec((HH, H), lambda i: (0, 0)),
        pl.BlockSpec((1, H), lambda i: (0, 0)),
        pl.BlockSpec((H, 1), lambda i: (0, 0)),
        pl.BlockSpec((1, 1), lambda i: (0, 0)),
    ],
    out_specs=pl.BlockSpec((NGRAPH, 1), lambda i: (0, 0)),
    out_shape=jax.ShapeDtypeStruct((NGRAPH, 1), jnp.float32),
)


def kernel(z, pos, batch, edge_index, emb, mw1_0, mb1_0, mw2_0, mb2_0, l1w_0,
           l2w_0, l2b_0, lw_0, lb_0, mw1_1, mb1_1, mw2_1, mb2_1, l1w_1, l2w_1,
           l2b_1, lw_1, lb_1, fl1w, fl1b, fl2w, fl2b, pw, pb):
    src = edge_index[0]
    dst = edge_index[1]
    posx = pos[:, 0]
    posy = pos[:, 1]
    posz = pos[:, 2]
    z2 = z.reshape(N, 1).astype(jnp.int32)
    batch2 = batch.reshape(N, 1).astype(jnp.int32)

    d2 = _dist2(posx, posy, posz, src, dst)
    d2r = jnp.pad(d2, (0, EP - E)).reshape(ER, 128)
    wf0 = _filter(d2r, mw1_0, mb1_0.reshape(1, H), mw2_0, mb2_0.reshape(1, H))
    wf1 = _filter(d2r, mw1_1, mb1_1.reshape(1, H), mw2_1, mb2_1.reshape(1, H))
    h0, xl0 = _embed(z2, emb, l1w_0)
    agg0 = _agg(xl0, wf0, src, dst)
    h1, xl1 = _update(agg0, h0, l2w_0, l2b_0.reshape(1, H), lw_0,
                      lb_0.reshape(1, H), l1w_1)
    agg1 = _agg(xl1, wf1, src, dst)
    out = _final(agg1, h1, batch2, l2w_1, l2b_1.reshape(1, H),
                 lw_1, lb_1.reshape(1, H), fl1w, fl1b.reshape(1, HH), fl2w,
                 fl2b.reshape(1, H), pw, pb.reshape(1, 1))
    return out


# speedup vs baseline: 1.6137x; 1.6137x over previous
import functools

import jax
import jax.numpy as jnp
import numpy as np
from jax import lax
from jax.experimental import pallas as pl
from jax.experimental.pallas import tpu as pltpu
from jax.experimental.pallas import tpu_sc as plsc

N = 10000
E = 320000
H = 128
HH = H // 2
NGAUSS = 10
NGRAPH = 64
CUTOFF = 10.0

NC = 2
NS = 16
LANES = 16

_MESH = plsc.VectorSubcoreMesh(
    core_axis_name="c", subcore_axis_name="s", num_cores=NC, num_subcores=NS
)
_SC_PARAMS = pltpu.CompilerParams(needs_layout_passes=False,
                                  use_tc_tiling_on_sc=False)

_STEP = np.float32(CUTOFF / (NGAUSS - 1))
_COEFF = np.float32(-0.5) / _STEP**2
_LOG2 = np.float32(np.log(2.0))


def _ssp(x):
    return jnp.maximum(x, 0.0) + jnp.log1p(jnp.exp(-jnp.abs(x))) - _LOG2


EPW = E // (NC * NS)


def _dist2_body(px_h, py_h, pz_h, src_h, dst_h, out_h, px, py, pz, sv, dv, ov):
    c = lax.axis_index("c")
    s = lax.axis_index("s")
    wid = s * NC + c
    base = wid * EPW
    pltpu.sync_copy(px_h, px)
    pltpu.sync_copy(py_h, py)
    pltpu.sync_copy(pz_h, pz)
    pltpu.sync_copy(src_h.at[pl.ds(base, EPW)], sv)
    pltpu.sync_copy(dst_h.at[pl.ds(base, EPW)], dv)

    def body(i, carry):
        si = sv[pl.ds(i * LANES, LANES)]
        di = dv[pl.ds(i * LANES, LANES)]
        dx = plsc.load_gather(px, [si]) - plsc.load_gather(px, [di])
        dy = plsc.load_gather(py, [si]) - plsc.load_gather(py, [di])
        dz = plsc.load_gather(pz, [si]) - plsc.load_gather(pz, [di])
        ov[pl.ds(i * LANES, LANES)] = dx * dx + dy * dy + dz * dz
        return carry

    lax.fori_loop(0, EPW // LANES, body, 0)
    pltpu.sync_copy(ov, out_h.at[pl.ds(base, EPW)])


_dist2 = pl.kernel(
    _dist2_body,
    out_type=jax.ShapeDtypeStruct((E,), jnp.float32),
    mesh=_MESH,
    compiler_params=_SC_PARAMS,
    scratch_types=[
        pltpu.VMEM((N,), jnp.float32),
        pltpu.VMEM((N,), jnp.float32),
        pltpu.VMEM((N,), jnp.float32),
        pltpu.VMEM((EPW,), jnp.int32),
        pltpu.VMEM((EPW,), jnp.int32),
        pltpu.VMEM((EPW,), jnp.float32),
    ],
)


K = 80
EPT = E // NS
NCHUNK = EPT // K
RPT = N // NS
ZR = 125


def _agg_body(xl_h, wf_h, src_h, dst_h, agg_h,
              xl_s, agg_s, sv, dv, gv, wv, zv,
              sem_is, sem_id, sem_g, sem_w, sem_s):
    c = lax.axis_index("c")
    s = lax.axis_index("s")
    row0 = s * RPT
    base = s * EPT
    cols = pl.ds(c * HH, HH)

    def zbody(i, carry):
        zero = jnp.zeros((LANES,), jnp.float32)
        for j in range(HH // LANES):
            zv[i, pl.ds(j * LANES, LANES)] = zero
        return carry

    lax.fori_loop(0, ZR, zbody, 0)
    for r in range(RPT // ZR):
        pltpu.sync_copy(zv, agg_s.at[pl.ds(row0 + r * ZR, ZR), :])

    pltpu.sync_copy(xl_h.at[pl.ds(row0, RPT), cols],
                    xl_s.at[pl.ds(row0, RPT), :])
    plsc.subcore_barrier()

    def issue_idx(t, b4):
        e0 = base + t * K
        pltpu.async_copy(src_h.at[pl.ds(e0, K)], sv.at[t % 2], sem_is.at[t % 2])
        pltpu.async_copy(dst_h.at[pl.ds(e0, K)], dv.at[b4], sem_id.at[b4])

    def wait_idx(t, b4):
        e0 = base + t * K
        pltpu.make_async_copy(src_h.at[pl.ds(e0, K)], sv.at[t % 2],
                              sem_is.at[t % 2]).wait()
        pltpu.make_async_copy(dst_h.at[pl.ds(e0, K)], dv.at[b4],
                              sem_id.at[b4]).wait()

    def issue_fetch(t, b):
        e0 = base + t * K
        pltpu.async_copy(xl_s.at[sv.at[b]], gv.at[b], sem_g.at[b])
        pltpu.async_copy(wf_h.at[pl.ds(e0, K), cols], wv.at[b], sem_w.at[b])

    def wait_fetch(t, b):
        e0 = base + t * K
        pltpu.make_async_copy(xl_s.at[sv.at[b]], gv.at[b], sem_g.at[b]).wait()
        pltpu.make_async_copy(wf_h.at[pl.ds(e0, K), cols], wv.at[b],
                              sem_w.at[b]).wait()

    def mul(b):
        def body(i, c2):
            for j2 in range(HH // LANES):
                sl = pl.ds(j2 * LANES, LANES)
                gv[b, i, sl] = gv[b, i, sl] * wv[b, i, sl]
            return c2

        lax.fori_loop(0, K, body, 0)

    def issue_scatter(b, b4):
        pltpu.async_copy(gv.at[b], agg_s.at[dv.at[b4]], sem_s.at[b], add=True)

    def wait_scatter(b, b4):
        pltpu.make_async_copy(gv.at[b], agg_s.at[dv.at[b4]],
                              sem_s.at[b]).wait()

    def substep(t, first=False, nm1=False, last=False):
        b, b4 = t % 2, t % 4
        wait_fetch(t, b)
        mul(b)
        issue_scatter(b, b4)
        if not last:
            wait_idx(t + 1, (t + 1) % 4)
        if not first:
            wait_scatter(1 - b, (t - 1) % 4)
        if not last:
            issue_fetch(t + 1, 1 - b)
        if not (nm1 or last):
            issue_idx(t + 2, (t + 2) % 4)

    issue_idx(0, 0)
    issue_idx(1, 1)
    wait_idx(0, 0)
    issue_fetch(0, 0)
    substep(0, first=True)
    substep(1)

    def quad(v, carry):
        t0 = 2 + 4 * v
        for q in range(4):
            substep(t0 + q)
        return carry

    lax.fori_loop(0, (NCHUNK - 6) // 4, quad, 0)
    substep(NCHUNK - 4)
    substep(NCHUNK - 3)
    substep(NCHUNK - 2, nm1=True)
    substep(NCHUNK - 1, last=True)
    wait_scatter((NCHUNK - 1) % 2, (NCHUNK - 1) % 4)

    plsc.subcore_barrier()
    pltpu.sync_copy(agg_s.at[pl.ds(row0, RPT), :],
                    agg_h.at[pl.ds(row0, RPT), cols])


_agg = pl.kernel(
    _agg_body,
    out_type=jax.ShapeDtypeStruct((N, H), jnp.float32),
    mesh=_MESH,
    compiler_params=_SC_PARAMS,
    scratch_types=[
        pltpu.VMEM_SHARED((N, HH), jnp.float32),
        pltpu.VMEM_SHARED((N, HH), jnp.float32),
        pltpu.VMEM((2, K), jnp.int32),
        pltpu.VMEM((4, K), jnp.int32),
        pltpu.VMEM((2, K, HH), jnp.float32),
        pltpu.VMEM((2, K, HH), jnp.float32),
        pltpu.VMEM((ZR, HH), jnp.float32),
        pltpu.SemaphoreType.DMA((2,)),
        pltpu.SemaphoreType.DMA((4,)),
        pltpu.SemaphoreType.DMA((2,)),
        pltpu.SemaphoreType.DMA((2,)),
        pltpu.SemaphoreType.DMA((2,)),
    ],
)


EP = 327680
TE = 2048
TB = TE // 128
GE = EP // TE
ER = EP // 128
TN = 2000
GN = N // TN


def _filter_body(d2_ref, mw1_ref, mb1_ref, mw2_ref, mb2_ref, wf_ref):
    d2d = d2_ref[...]
    distd = jnp.sqrt(d2d + 1e-12)
    cenvd = 0.5 * (jnp.cos(distd * jnp.pi / CUTOFF) + 1.0)
    offc = (lax.broadcasted_iota(jnp.int32, (NGAUSS, 128), 0)
            .astype(jnp.float32) * _STEP)
    mw1 = mw1_ref[...]
    rows = []
    for r in range(TB):
        distb = jnp.broadcast_to(distd[r:r + 1, :], (NGAUSS, 128))
        rbf_r = jnp.exp(_COEFF * (distb - offc) ** 2)
        t_r = lax.dot_general(rbf_r, mw1, (((0,), (0,)), ((), ())),
                              preferred_element_type=jnp.float32)
        rows.append(t_r[None])
    t = jnp.concatenate(rows, axis=0)
    cenv3 = jnp.broadcast_to(cenvd[:, :, None], (TB, 128, H))
    s = _ssp(t + mb1_ref[...].reshape(1, 1, H)) * cenv3
    wf = jnp.dot(s.reshape(TE, H), mw2_ref[...],
                 preferred_element_type=jnp.float32)
    wf = wf + (cenv3 * mb2_ref[...].reshape(1, 1, H)).reshape(TE, H)
    wf_ref[...] = wf


_filter = pl.pallas_call(
    _filter_body,
    grid=(GE,),
    in_specs=[
        pl.BlockSpec((TB, 128), lambda i: (i, 0)),
        pl.BlockSpec((NGAUSS, H), lambda i: (0, 0)),
        pl.BlockSpec((1, H), lambda i: (0, 0)),
        pl.BlockSpec((H, H), lambda i: (0, 0)),
        pl.BlockSpec((1, H), lambda i: (0, 0)),
    ],
    out_specs=pl.BlockSpec((TE, H), lambda i: (i, 0)),
    out_shape=jax.ShapeDtypeStruct((EP, H), jnp.float32),
)


def _embed_body(z_ref, emb_ref, l1w_ref, h_ref, xl_ref):
    z = z_ref[...]
    oh = (z == lax.broadcasted_iota(jnp.int32, (TN, 100), 1)).astype(jnp.float32)
    h = jnp.dot(oh, emb_ref[...], preferred_element_type=jnp.float32)
    xl = jnp.dot(h, l1w_ref[...], preferred_element_type=jnp.float32)
    h_ref[...] = h
    xl_ref[...] = xl


_embed = pl.pallas_call(
    _embed_body,
    grid=(GN,),
    in_specs=[
        pl.BlockSpec((TN, 1), lambda i: (i, 0)),
        pl.BlockSpec((100, H), lambda i: (0, 0)),
        pl.BlockSpec((H, H), lambda i: (0, 0)),
    ],
    out_specs=[
        pl.BlockSpec((TN, H), lambda i: (i, 0)),
        pl.BlockSpec((TN, H), lambda i: (i, 0)),
    ],
    out_shape=[
        jax.ShapeDtypeStruct((N, H), jnp.float32),
        jax.ShapeDtypeStruct((N, H), jnp.float32),
    ],
)


def _update_body(agg_ref, h_ref, l2w_ref, l2b_ref, lw_ref, lb_ref,
                 l1wn_ref, hn_ref, xl_ref):
    x = jnp.dot(agg_ref[...], l2w_ref[...], preferred_element_type=jnp.float32)
    x = _ssp(x + l2b_ref[...])
    x = jnp.dot(x, lw_ref[...], preferred_element_type=jnp.float32) + lb_ref[...]
    hn = h_ref[...] + x
    xl = jnp.dot(hn, l1wn_ref[...], preferred_element_type=jnp.float32)
    hn_ref[...] = hn
    xl_ref[...] = xl


_update = pl.pallas_call(
    _update_body,
    grid=(GN,),
    in_specs=[
        pl.BlockSpec((TN, H), lambda i: (i, 0)),
        pl.BlockSpec((TN, H), lambda i: (i, 0)),
        pl.BlockSpec((H, H), lambda i: (0, 0)),
        pl.BlockSpec((1, H), lambda i: (0, 0)),
        pl.BlockSpec((H, H), lambda i: (0, 0)),
        pl.BlockSpec((1, H), lambda i: (0, 0)),
        pl.BlockSpec((H, H), lambda i: (0, 0)),
    ],
    out_specs=[
        pl.BlockSpec((TN, H), lambda i: (i, 0)),
        pl.BlockSpec((TN, H), lambda i: (i, 0)),
    ],
    out_shape=[
        jax.ShapeDtypeStruct((N, H), jnp.float32),
        jax.ShapeDtypeStruct((N, H), jnp.float32),
    ],
)


def _final_body(agg_ref, h_ref, batch_ref, l2w_ref, l2b_ref, lw_ref,
                lb_ref, fl1w_ref, fl1b_ref, fl2w_ref, fl2b_ref, pw_ref, pb_ref,
                out_ref):
    i = pl.program_id(0)
    x = jnp.dot(agg_ref[...], l2w_ref[...], preferred_element_type=jnp.float32)
    x = _ssp(x + l2b_ref[...])
    x = jnp.dot(x, lw_ref[...], preferred_element_type=jnp.float32) + lb_ref[...]
    h2 = h_ref[...] + x
    hf = _ssp(jnp.dot(h2, fl1w_ref[...], preferred_element_type=jnp.float32)
              + fl1b_ref[...])
    hf = jnp.dot(hf, fl2w_ref[...], preferred_element_type=jnp.float32)
    hf = hf + fl2b_ref[...]
    hp = jnp.dot(hf, pw_ref[...], preferred_element_type=jnp.float32)
    oh = (batch_ref[...] == lax.broadcasted_iota(jnp.int32, (TN, NGRAPH), 1))
    part = lax.dot_general(oh.astype(jnp.float32), hp,
                           (((0,), (0,)), ((), ())),
                           preferred_element_type=jnp.float32)

    @pl.when(i == 0)
    def _():
        out_ref[...] = part + pb_ref[...]

    @pl.when(i > 0)
    def _():
        out_ref[...] = out_ref[...] + part


_final = pl.pallas_call(
    _final_body,
    grid=(GN,),
    in_specs=[
        pl.BlockSpec((TN, H), lambda i: (i, 0)),
        pl.BlockSpec((TN, H), lambda i: (i, 0)),
        pl.BlockSpec((TN, 1), lambda i: (i, 0)),
        pl.BlockSpec((H, H), lambda i: (0, 0)),
        pl.BlockSpec((1, H), lambda i: (0, 0)),
        pl.BlockSpec((H, H), lambda i: (0, 0)),
        pl.BlockSpec((1, H), lambda i: (0, 0)),
        pl.BlockSpec((H, HH), lambda i: (0, 0)),
        pl.BlockSpec((1, HH), lambda i: (0, 0)),
        pl.BlockSpec((HH, H), lambda i: (0, 0)),
        pl.BlockSpec((1, H), lambda i: (0, 0)),
        pl.BlockSpec((H, 1), lambda i: (0, 0)),
        pl.BlockSpec((1, 1), lambda i: (0, 0)),
    ],
    out_specs=pl.BlockSpec((NGRAPH, 1), lambda i: (0, 0)),
    out_shape=jax.ShapeDtypeStruct((NGRAPH, 1), jnp.float32),
)


def kernel(z, pos, batch, edge_index, emb, mw1_0, mb1_0, mw2_0, mb2_0, l1w_0,
           l2w_0, l2b_0, lw_0, lb_0, mw1_1, mb1_1, mw2_1, mb2_1, l1w_1, l2w_1,
           l2b_1, lw_1, lb_1, fl1w, fl1b, fl2w, fl2b, pw, pb):
    src = edge_index[0]
    dst = edge_index[1]
    posx = pos[:, 0]
    posy = pos[:, 1]
    posz = pos[:, 2]
    z2 = z.reshape(N, 1).astype(jnp.int32)
    batch2 = batch.reshape(N, 1).astype(jnp.int32)

    d2 = _dist2(posx, posy, posz, src, dst)
    d2r = jnp.pad(d2, (0, EP - E)).reshape(ER, 128)
    wf0 = _filter(d2r, mw1_0, mb1_0.reshape(1, H), mw2_0, mb2_0.reshape(1, H))
    wf1 = _filter(d2r, mw1_1, mb1_1.reshape(1, H), mw2_1, mb2_1.reshape(1, H))
    h0, xl0 = _embed(z2, emb, l1w_0)
    agg0 = _agg(xl0, wf0, src, dst)
    h1, xl1 = _update(agg0, h0, l2w_0, l2b_0.reshape(1, H), lw_0,
                      lb_0.reshape(1, H), l1w_1)
    agg1 = _agg(xl1, wf1, src, dst)
    out = _final(agg1, h1, batch2, l2w_1, l2b_1.reshape(1, H),
                 lw_1, lb_1.reshape(1, H), fl1w, fl1b.reshape(1, HH), fl2w,
                 fl2b.reshape(1, H), pw, pb.reshape(1, 1))
    return out


# speedup vs baseline: 1.6908x; 1.0478x over previous
import functools

import jax
import jax.numpy as jnp
import numpy as np
from jax import lax
from jax.experimental import pallas as pl
from jax.experimental.pallas import tpu as pltpu
from jax.experimental.pallas import tpu_sc as plsc

N = 10000
E = 320000
H = 128
HH = H // 2
NGAUSS = 10
NGRAPH = 64
CUTOFF = 10.0

NC = 2
NS = 16
LANES = 16

_MESH = plsc.VectorSubcoreMesh(
    core_axis_name="c", subcore_axis_name="s", num_cores=NC, num_subcores=NS
)
_SC_PARAMS = pltpu.CompilerParams(needs_layout_passes=False,
                                  use_tc_tiling_on_sc=False)

_STEP = np.float32(CUTOFF / (NGAUSS - 1))
_COEFF = np.float32(-0.5) / _STEP**2
_LOG2 = np.float32(np.log(2.0))


def _ssp(x):
    return jnp.maximum(x, 0.0) + jnp.log1p(jnp.exp(-jnp.abs(x))) - _LOG2


EPW = E // (NC * NS)


def _dist2_body(px_h, py_h, pz_h, src_h, dst_h, out_h, px, py, pz, sv, dv, ov):
    c = lax.axis_index("c")
    s = lax.axis_index("s")
    wid = s * NC + c
    base = wid * EPW
    pltpu.sync_copy(px_h, px)
    pltpu.sync_copy(py_h, py)
    pltpu.sync_copy(pz_h, pz)
    pltpu.sync_copy(src_h.at[pl.ds(base, EPW)], sv)
    pltpu.sync_copy(dst_h.at[pl.ds(base, EPW)], dv)

    def body(i, carry):
        si = sv[pl.ds(i * LANES, LANES)]
        di = dv[pl.ds(i * LANES, LANES)]
        dx = plsc.load_gather(px, [si]) - plsc.load_gather(px, [di])
        dy = plsc.load_gather(py, [si]) - plsc.load_gather(py, [di])
        dz = plsc.load_gather(pz, [si]) - plsc.load_gather(pz, [di])
        ov[pl.ds(i * LANES, LANES)] = dx * dx + dy * dy + dz * dz
        return carry

    lax.fori_loop(0, EPW // LANES, body, 0)
    pltpu.sync_copy(ov, out_h.at[pl.ds(base, EPW)])


_dist2 = pl.kernel(
    _dist2_body,
    out_type=jax.ShapeDtypeStruct((E,), jnp.float32),
    mesh=_MESH,
    compiler_params=_SC_PARAMS,
    scratch_types=[
        pltpu.VMEM((N,), jnp.float32),
        pltpu.VMEM((N,), jnp.float32),
        pltpu.VMEM((N,), jnp.float32),
        pltpu.VMEM((EPW,), jnp.int32),
        pltpu.VMEM((EPW,), jnp.int32),
        pltpu.VMEM((EPW,), jnp.float32),
    ],
)


K = 80
EPT = E // NS
NCHUNK = EPT // K
RPT = N // NS
ZR = 125


def _agg_body(xl_h, wf_h, src_h, dst_h, agg_h,
              xl_s, agg_s, sv, dv, gv, wv, zv,
              sem_is, sem_id, sem_g, sem_w):
    c = lax.axis_index("c")
    s = lax.axis_index("s")
    row0 = s * RPT
    base = s * EPT
    cols = pl.ds(c * HH, HH)

    def zbody(i, carry):
        zero = jnp.zeros((LANES,), jnp.float32)
        for j in range(HH // LANES):
            zv[i, pl.ds(j * LANES, LANES)] = zero
        return carry

    lax.fori_loop(0, ZR, zbody, 0)
    for r in range(RPT // ZR):
        pltpu.sync_copy(zv, agg_s.at[pl.ds(row0 + r * ZR, ZR), :])

    pltpu.sync_copy(xl_h.at[pl.ds(row0, RPT), cols],
                    xl_s.at[pl.ds(row0, RPT), :])
    plsc.subcore_barrier()

    def issue_idx(t, b):
        e0 = base + t * K
        pltpu.async_copy(src_h.at[pl.ds(e0, K)], sv.at[b], sem_is.at[b])
        pltpu.async_copy(dst_h.at[pl.ds(e0, K)], dv.at[b], sem_id.at[b])

    def wait_idx(t, b):
        e0 = base + t * K
        pltpu.make_async_copy(src_h.at[pl.ds(e0, K)], sv.at[b],
                              sem_is.at[b]).wait()
        pltpu.make_async_copy(dst_h.at[pl.ds(e0, K)], dv.at[b],
                              sem_id.at[b]).wait()

    def issue_fetch(t, b):
        e0 = base + t * K
        pltpu.async_copy(xl_s.at[sv.at[b]], gv.at[b], sem_g.at[b])
        pltpu.async_copy(wf_h.at[pl.ds(e0, K), cols], wv.at[b], sem_w.at[b])

    def wait_fetch(t, b):
        e0 = base + t * K
        pltpu.make_async_copy(xl_s.at[sv.at[b]], gv.at[b], sem_g.at[b]).wait()
        pltpu.make_async_copy(wf_h.at[pl.ds(e0, K), cols], wv.at[b],
                              sem_w.at[b]).wait()

    def compute_scatter(b):
        def mul(i, c2):
            for j2 in range(HH // LANES):
                sl = pl.ds(j2 * LANES, LANES)
                gv[b, i, sl] = gv[b, i, sl] * wv[b, i, sl]
            return c2

        lax.fori_loop(0, K, mul, 0)
        pltpu.sync_copy(gv.at[b], agg_s.at[dv.at[b]], add=True)

    issue_idx(0, 0)
    issue_idx(1, 1)
    wait_idx(0, 0)
    issue_fetch(0, 0)

    def step(u, carry):
        for b, toff in ((0, 0), (1, 1)):
            t = 2 * u + toff
            wait_idx(t + 1, 1 - b)
            issue_fetch(t + 1, 1 - b)
            wait_fetch(t, b)
            compute_scatter(b)
            issue_idx(t + 2, b)
        return carry

    lax.fori_loop(0, NCHUNK // 2 - 1, step, 0)
    tlast = NCHUNK - 2
    wait_fetch(tlast, 0)
    compute_scatter(0)
    wait_idx(tlast + 1, 1)
    issue_fetch(tlast + 1, 1)
    wait_fetch(tlast + 1, 1)
    compute_scatter(1)

    plsc.subcore_barrier()
    pltpu.sync_copy(agg_s.at[pl.ds(row0, RPT), :],
                    agg_h.at[pl.ds(row0, RPT), cols])


_agg = pl.kernel(
    _agg_body,
    out_type=jax.ShapeDtypeStruct((N, H), jnp.float32),
    mesh=_MESH,
    compiler_params=_SC_PARAMS,
    scratch_types=[
        pltpu.VMEM_SHARED((N, HH), jnp.float32),
        pltpu.VMEM_SHARED((N, HH), jnp.float32),
        pltpu.VMEM((2, K), jnp.int32),
        pltpu.VMEM((2, K), jnp.int32),
        pltpu.VMEM((2, K, HH), jnp.float32),
        pltpu.VMEM((2, K, HH), jnp.float32),
        pltpu.VMEM((ZR, HH), jnp.float32),
        pltpu.SemaphoreType.DMA((2,)),
        pltpu.SemaphoreType.DMA((2,)),
        pltpu.SemaphoreType.DMA((2,)),
        pltpu.SemaphoreType.DMA((2,)),
    ],
)


EP = 327680
TE = 2048
TB = TE // 128
GE = EP // TE
ER = EP // 128
TN = 2000
GN = N // TN


def _filter_body(d2_ref, mw1_ref, mb1_ref, mw2_ref, mb2_ref, wf_ref):
    d2d = d2_ref[...]
    distd = jnp.sqrt(d2d + 1e-12)
    cenvd = 0.5 * (jnp.cos(distd * jnp.pi / CUTOFF) + 1.0)
    offc = (lax.broadcasted_iota(jnp.int32, (NGAUSS, 128), 0)
            .astype(jnp.float32) * _STEP)
    mw1 = mw1_ref[...]
    rows = []
    for r in range(TB):
        distb = jnp.broadcast_to(distd[r:r + 1, :], (NGAUSS, 128))
        rbf_r = jnp.exp(_COEFF * (distb - offc) ** 2)
        t_r = lax.dot_general(rbf_r, mw1, (((0,), (0,)), ((), ())),
                              preferred_element_type=jnp.float32)
        rows.append(t_r[None])
    t = jnp.concatenate(rows, axis=0)
    cenv3 = jnp.broadcast_to(cenvd[:, :, None], (TB, 128, H))
    s = _ssp(t + mb1_ref[...].reshape(1, 1, H)) * cenv3
    wf = jnp.dot(s.reshape(TE, H), mw2_ref[...],
                 preferred_element_type=jnp.float32)
    wf = wf + (cenv3 * mb2_ref[...].reshape(1, 1, H)).reshape(TE, H)
    wf_ref[...] = wf


_filter = pl.pallas_call(
    _filter_body,
    grid=(GE,),
    in_specs=[
        pl.BlockSpec((TB, 128), lambda i: (i, 0)),
        pl.BlockSpec((NGAUSS, H), lambda i: (0, 0)),
        pl.BlockSpec((1, H), lambda i: (0, 0)),
        pl.BlockSpec((H, H), lambda i: (0, 0)),
        pl.BlockSpec((1, H), lambda i: (0, 0)),
    ],
    out_specs=pl.BlockSpec((TE, H), lambda i: (i, 0)),
    out_shape=jax.ShapeDtypeStruct((EP, H), jnp.float32),
)


def _embed_body(z_ref, emb_ref, l1w_ref, h_ref, xl_ref):
    z = z_ref[...]
    oh = (z == lax.broadcasted_iota(jnp.int32, (TN, 100), 1)).astype(jnp.float32)
    h = jnp.dot(oh, emb_ref[...], preferred_element_type=jnp.float32)
    xl = jnp.dot(h, l1w_ref[...], preferred_element_type=jnp.float32)
    h_ref[...] = h
    xl_ref[...] = xl


_embed = pl.pallas_call(
    _embed_body,
    grid=(GN,),
    in_specs=[
        pl.BlockSpec((TN, 1), lambda i: (i, 0)),
        pl.BlockSpec((100, H), lambda i: (0, 0)),
        pl.BlockSpec((H, H), lambda i: (0, 0)),
    ],
    out_specs=[
        pl.BlockSpec((TN, H), lambda i: (i, 0)),
        pl.BlockSpec((TN, H), lambda i: (i, 0)),
    ],
    out_shape=[
        jax.ShapeDtypeStruct((N, H), jnp.float32),
        jax.ShapeDtypeStruct((N, H), jnp.float32),
    ],
)


def _update_body(agg_ref, h_ref, l2w_ref, l2b_ref, lw_ref, lb_ref,
                 l1wn_ref, hn_ref, xl_ref):
    x = jnp.dot(agg_ref[...], l2w_ref[...], preferred_element_type=jnp.float32)
    x = _ssp(x + l2b_ref[...])
    x = jnp.dot(x, lw_ref[...], preferred_element_type=jnp.float32) + lb_ref[...]
    hn = h_ref[...] + x
    xl = jnp.dot(hn, l1wn_ref[...], preferred_element_type=jnp.float32)
    hn_ref[...] = hn
    xl_ref[...] = xl


_update = pl.pallas_call(
    _update_body,
    grid=(GN,),
    in_specs=[
        pl.BlockSpec((TN, H), lambda i: (i, 0)),
        pl.BlockSpec((TN, H), lambda i: (i, 0)),
        pl.BlockSpec((H, H), lambda i: (0, 0)),
        pl.BlockSpec((1, H), lambda i: (0, 0)),
        pl.BlockSpec((H, H), lambda i: (0, 0)),
        pl.BlockSpec((1, H), lambda i: (0, 0)),
        pl.BlockSpec((H, H), lambda i: (0, 0)),
    ],
    out_specs=[
        pl.BlockSpec((TN, H), lambda i: (i, 0)),
        pl.BlockSpec((TN, H), lambda i: (i, 0)),
    ],
    out_shape=[
        jax.ShapeDtypeStruct((N, H), jnp.float32),
        jax.ShapeDtypeStruct((N, H), jnp.float32),
    ],
)


def _final_body(agg_ref, h_ref, batch_ref, l2w_ref, l2b_ref, lw_ref,
                lb_ref, fl1w_ref, fl1b_ref, fl2w_ref, fl2b_ref, pw_ref, pb_ref,
                out_ref):
    i = pl.program_id(0)
    x = jnp.dot(agg_ref[...], l2w_ref[...], preferred_element_type=jnp.float32)
    x = _ssp(x + l2b_ref[...])
    x = jnp.dot(x, lw_ref[...], preferred_element_type=jnp.float32) + lb_ref[...]
    h2 = h_ref[...] + x
    hf = _ssp(jnp.dot(h2, fl1w_ref[...], preferred_element_type=jnp.float32)
              + fl1b_ref[...])
    hf = jnp.dot(hf, fl2w_ref[...], preferred_element_type=jnp.float32)
    hf = hf + fl2b_ref[...]
    hp = jnp.dot(hf, pw_ref[...], preferred_element_type=jnp.float32)
    oh = (batch_ref[...] == lax.broadcasted_iota(jnp.int32, (TN, NGRAPH), 1))
    part = lax.dot_general(oh.astype(jnp.float32), hp,
                           (((0,), (0,)), ((), ())),
                           preferred_element_type=jnp.float32)

    @pl.when(i == 0)
    def _():
        out_ref[...] = part + pb_ref[...]

    @pl.when(i > 0)
    def _():
        out_ref[...] = out_ref[...] + part


_final = pl.pallas_call(
    _final_body,
    grid=(GN,),
    in_specs=[
        pl.BlockSpec((TN, H), lambda i: (i, 0)),
        pl.BlockSpec((TN, H), lambda i: (i, 0)),
        pl.BlockSpec((TN, 1), lambda i: (i, 0)),
        pl.BlockSpec((H, H), lambda i: (0, 0)),
        pl.BlockSpec((1, H), lambda i: (0, 0)),
        pl.BlockSpec((H, H), lambda i: (0, 0)),
        pl.BlockSpec((1, H), lambda i: (0, 0)),
        pl.BlockSpec((H, HH), lambda i: (0, 0)),
        pl.BlockSpec((1, HH), lambda i: (0, 0)),
        pl.BlockSpec((HH, H), lambda i: (0, 0)),
        pl.BlockSpec((1, H), lambda i: (0, 0)),
        pl.BlockSpec((H, 1), lambda i: (0, 0)),
        pl.BlockSpec((1, 1), lambda i: (0, 0)),
    ],
    out_specs=pl.BlockSpec((NGRAPH, 1), lambda i: (0, 0)),
    out_shape=jax.ShapeDtypeStruct((NGRAPH, 1), jnp.float32),
)


def kernel(z, pos, batch, edge_index, emb, mw1_0, mb1_0, mw2_0, mb2_0, l1w_0,
           l2w_0, l2b_0, lw_0, lb_0, mw1_1, mb1_1, mw2_1, mb2_1, l1w_1, l2w_1,
           l2b_1, lw_1, lb_1, fl1w, fl1b, fl2w, fl2b, pw, pb):
    src = edge_index[0]
    dst = edge_index[1]
    posx = pos[:, 0]
    posy = pos[:, 1]
    posz = pos[:, 2]
    z2 = z.reshape(N, 1).astype(jnp.int32)
    batch2 = batch.reshape(N, 1).astype(jnp.int32)

    d2 = _dist2(posx, posy, posz, src, dst)
    d2r = jnp.pad(d2, (0, EP - E)).reshape(ER, 128)
    wf0 = _filter(d2r, mw1_0, mb1_0.reshape(1, H), mw2_0, mb2_0.reshape(1, H))
    wf1 = _filter(d2r, mw1_1, mb1_1.reshape(1, H), mw2_1, mb2_1.reshape(1, H))
    h0, xl0 = _embed(z2, emb, l1w_0)
    agg0 = _agg(xl0, wf0, src, dst)
    h1, xl1 = _update(agg0, h0, l2w_0, l2b_0.reshape(1, H), lw_0,
                      lb_0.reshape(1, H), l1w_1)
    agg1 = _agg(xl1, wf1, src, dst)
    out = _final(agg1, h1, batch2, l2w_1, l2b_1.reshape(1, H),
                 lw_1, lb_1.reshape(1, H), fl1w, fl1b.reshape(1, HH), fl2w,
                 fl2b.reshape(1, H), pw, pb.reshape(1, 1))
    return out


# speedup vs baseline: 1.9727x; 1.1667x over previous
import functools

import jax
import jax.numpy as jnp
import numpy as np
from jax import lax
from jax.experimental import pallas as pl
from jax.experimental.pallas import tpu as pltpu
from jax.experimental.pallas import tpu_sc as plsc

N = 10000
E = 320000
H = 128
HH = H // 2
NGAUSS = 10
NGRAPH = 64
CUTOFF = 10.0

NC = 2
NS = 16
LANES = 16

_MESH = plsc.VectorSubcoreMesh(
    core_axis_name="c", subcore_axis_name="s", num_cores=NC, num_subcores=NS
)
_SC_PARAMS = pltpu.CompilerParams(needs_layout_passes=False,
                                  use_tc_tiling_on_sc=False)

_STEP = np.float32(CUTOFF / (NGAUSS - 1))
_COEFF = np.float32(-0.5) / _STEP**2
_LOG2 = np.float32(np.log(2.0))


def _ssp(x):
    return jnp.maximum(x, 0.0) + jnp.log1p(jnp.exp(-jnp.abs(x))) - _LOG2


EPW = E // (NC * NS)


def _dist2_body(px_h, py_h, pz_h, src_h, dst_h, out_h, px, py, pz, sv, dv, ov):
    c = lax.axis_index("c")
    s = lax.axis_index("s")
    wid = s * NC + c
    base = wid * EPW
    pltpu.sync_copy(px_h, px)
    pltpu.sync_copy(py_h, py)
    pltpu.sync_copy(pz_h, pz)
    pltpu.sync_copy(src_h.at[pl.ds(base, EPW)], sv)
    pltpu.sync_copy(dst_h.at[pl.ds(base, EPW)], dv)

    def body(i, carry):
        si = sv[pl.ds(i * LANES, LANES)]
        di = dv[pl.ds(i * LANES, LANES)]
        dx = plsc.load_gather(px, [si]) - plsc.load_gather(px, [di])
        dy = plsc.load_gather(py, [si]) - plsc.load_gather(py, [di])
        dz = plsc.load_gather(pz, [si]) - plsc.load_gather(pz, [di])
        ov[pl.ds(i * LANES, LANES)] = dx * dx + dy * dy + dz * dz
        return carry

    lax.fori_loop(0, EPW // LANES, body, 0)
    pltpu.sync_copy(ov, out_h.at[pl.ds(base, EPW)])


_dist2 = pl.kernel(
    _dist2_body,
    out_type=jax.ShapeDtypeStruct((E,), jnp.float32),
    mesh=_MESH,
    compiler_params=_SC_PARAMS,
    scratch_types=[
        pltpu.VMEM((N,), jnp.float32),
        pltpu.VMEM((N,), jnp.float32),
        pltpu.VMEM((N,), jnp.float32),
        pltpu.VMEM((EPW,), jnp.int32),
        pltpu.VMEM((EPW,), jnp.int32),
        pltpu.VMEM((EPW,), jnp.float32),
    ],
)


K = 80
EPT = E // (2 * NS)
NCHUNK = EPT // K
RPT = N // NS
ZR = 25


def _agg_body(xl_h, wf_h, src_h, dst_h, agg0_h, agg1_h,
              agg_s, sv, dv, gv, wv, zv,
              sem_is, sem_id, sem_g, sem_w):
    c = lax.axis_index("c")
    s = lax.axis_index("s")
    row0 = s * RPT
    base = c * (E // 2) + s * EPT

    def zbody(i, carry):
        zero = jnp.zeros((LANES,), jnp.float32)
        for j in range(H // LANES):
            zv[i, pl.ds(j * LANES, LANES)] = zero
        return carry

    lax.fori_loop(0, ZR, zbody, 0)
    for r in range(RPT // ZR):
        pltpu.sync_copy(zv, agg_s.at[pl.ds(row0 + r * ZR, ZR), :])
    plsc.subcore_barrier()

    def issue_idx(t, b):
        e0 = base + t * K
        pltpu.async_copy(src_h.at[pl.ds(e0, K)], sv.at[b], sem_is.at[b])
        pltpu.async_copy(dst_h.at[pl.ds(e0, K)], dv.at[b], sem_id.at[b])

    def wait_idx(t, b):
        e0 = base + t * K
        pltpu.make_async_copy(src_h.at[pl.ds(e0, K)], sv.at[b],
                              sem_is.at[b]).wait()
        pltpu.make_async_copy(dst_h.at[pl.ds(e0, K)], dv.at[b],
                              sem_id.at[b]).wait()

    def issue_fetch(t, b):
        e0 = base + t * K
        pltpu.async_copy(xl_h.at[sv.at[b]], gv.at[b], sem_g.at[b])
        pltpu.async_copy(wf_h.at[pl.ds(e0, K), :], wv.at[b], sem_w.at[b])

    def wait_fetch(t, b):
        e0 = base + t * K
        pltpu.make_async_copy(xl_h.at[sv.at[b]], gv.at[b], sem_g.at[b]).wait()
        pltpu.make_async_copy(wf_h.at[pl.ds(e0, K), :], wv.at[b],
                              sem_w.at[b]).wait()

    def compute_scatter(b):
        def mul(i, c2):
            for j2 in range(H // LANES):
                sl = pl.ds(j2 * LANES, LANES)
                gv[b, i, sl] = gv[b, i, sl] * wv[b, i, sl]
            return c2

        lax.fori_loop(0, K, mul, 0)
        pltpu.sync_copy(gv.at[b], agg_s.at[dv.at[b]], add=True)

    issue_idx(0, 0)
    issue_idx(1, 1)
    wait_idx(0, 0)
    issue_fetch(0, 0)

    def step(u, carry):
        for b, toff in ((0, 0), (1, 1)):
            t = 2 * u + toff
            wait_idx(t + 1, 1 - b)
            issue_fetch(t + 1, 1 - b)
            wait_fetch(t, b)
            compute_scatter(b)
            issue_idx(t + 2, b)
        return carry

    lax.fori_loop(0, NCHUNK // 2, step, 0)
    tlast = NCHUNK - 1
    wait_fetch(tlast, 0)
    compute_scatter(0)
    wait_idx(tlast + 1, 1)

    plsc.subcore_barrier()

    @pl.when(c == 0)
    def _():
        pltpu.sync_copy(agg_s.at[pl.ds(row0, RPT), :],
                        agg0_h.at[pl.ds(row0, RPT), :])

    @pl.when(c == 1)
    def _():
        pltpu.sync_copy(agg_s.at[pl.ds(row0, RPT), :],
                        agg1_h.at[pl.ds(row0, RPT), :])


_agg = pl.kernel(
    _agg_body,
    out_type=(
        jax.ShapeDtypeStruct((N, H), jnp.float32),
        jax.ShapeDtypeStruct((N, H), jnp.float32),
    ),
    mesh=_MESH,
    compiler_params=_SC_PARAMS,
    scratch_types=[
        pltpu.VMEM_SHARED((N, H), jnp.float32),
        pltpu.VMEM((2, K), jnp.int32),
        pltpu.VMEM((2, K), jnp.int32),
        pltpu.VMEM((2, K, H), jnp.float32),
        pltpu.VMEM((2, K, H), jnp.float32),
        pltpu.VMEM((ZR, H), jnp.float32),
        pltpu.SemaphoreType.DMA((2,)),
        pltpu.SemaphoreType.DMA((2,)),
        pltpu.SemaphoreType.DMA((2,)),
        pltpu.SemaphoreType.DMA((2,)),
    ],
)


EP = 327680
TE = 2048
TB = TE // 128
GE = EP // TE
ER = EP // 128
TN = 2000
GN = N // TN


def _filter_body(d2_ref, mw1_ref, mb1_ref, mw2_ref, mb2_ref, wf_ref):
    d2d = d2_ref[...]
    distd = jnp.sqrt(d2d + 1e-12)
    cenvd = 0.5 * (jnp.cos(distd * jnp.pi / CUTOFF) + 1.0)
    offc = (lax.broadcasted_iota(jnp.int32, (NGAUSS, 128), 0)
            .astype(jnp.float32) * _STEP)
    mw1 = mw1_ref[...]
    rows = []
    for r in range(TB):
        distb = jnp.broadcast_to(distd[r:r + 1, :], (NGAUSS, 128))
        rbf_r = jnp.exp(_COEFF * (distb - offc) ** 2)
        t_r = lax.dot_general(rbf_r, mw1, (((0,), (0,)), ((), ())),
                              preferred_element_type=jnp.float32)
        rows.append(t_r[None])
    t = jnp.concatenate(rows, axis=0)
    cenv3 = jnp.broadcast_to(cenvd[:, :, None], (TB, 128, H))
    s = _ssp(t + mb1_ref[...].reshape(1, 1, H)) * cenv3
    wf = jnp.dot(s.reshape(TE, H), mw2_ref[...],
                 preferred_element_type=jnp.float32)
    wf = wf + (cenv3 * mb2_ref[...].reshape(1, 1, H)).reshape(TE, H)
    wf_ref[...] = wf


_filter = pl.pallas_call(
    _filter_body,
    grid=(GE,),
    in_specs=[
        pl.BlockSpec((TB, 128), lambda i: (i, 0)),
        pl.BlockSpec((NGAUSS, H), lambda i: (0, 0)),
        pl.BlockSpec((1, H), lambda i: (0, 0)),
        pl.BlockSpec((H, H), lambda i: (0, 0)),
        pl.BlockSpec((1, H), lambda i: (0, 0)),
    ],
    out_specs=pl.BlockSpec((TE, H), lambda i: (i, 0)),
    out_shape=jax.ShapeDtypeStruct((EP, H), jnp.float32),
)


def _embed_body(z_ref, emb_ref, l1w_ref, h_ref, xl_ref):
    z = z_ref[...]
    oh = (z == lax.broadcasted_iota(jnp.int32, (TN, 100), 1)).astype(jnp.float32)
    h = jnp.dot(oh, emb_ref[...], preferred_element_type=jnp.float32)
    xl = jnp.dot(h, l1w_ref[...], preferred_element_type=jnp.float32)
    h_ref[...] = h
    xl_ref[...] = xl


_embed = pl.pallas_call(
    _embed_body,
    grid=(GN,),
    in_specs=[
        pl.BlockSpec((TN, 1), lambda i: (i, 0)),
        pl.BlockSpec((100, H), lambda i: (0, 0)),
        pl.BlockSpec((H, H), lambda i: (0, 0)),
    ],
    out_specs=[
        pl.BlockSpec((TN, H), lambda i: (i, 0)),
        pl.BlockSpec((TN, H), lambda i: (i, 0)),
    ],
    out_shape=[
        jax.ShapeDtypeStruct((N, H), jnp.float32),
        jax.ShapeDtypeStruct((N, H), jnp.float32),
    ],
)


def _update_body(agga_ref, aggb_ref, h_ref, l2w_ref, l2b_ref, lw_ref, lb_ref,
                 l1wn_ref, hn_ref, xl_ref):
    agg = agga_ref[...] + aggb_ref[...]
    x = jnp.dot(agg, l2w_ref[...], preferred_element_type=jnp.float32)
    x = _ssp(x + l2b_ref[...])
    x = jnp.dot(x, lw_ref[...], preferred_element_type=jnp.float32) + lb_ref[...]
    hn = h_ref[...] + x
    xl = jnp.dot(hn, l1wn_ref[...], preferred_element_type=jnp.float32)
    hn_ref[...] = hn
    xl_ref[...] = xl


_update = pl.pallas_call(
    _update_body,
    grid=(GN,),
    in_specs=[
        pl.BlockSpec((TN, H), lambda i: (i, 0)),
        pl.BlockSpec((TN, H), lambda i: (i, 0)),
        pl.BlockSpec((TN, H), lambda i: (i, 0)),
        pl.BlockSpec((H, H), lambda i: (0, 0)),
        pl.BlockSpec((1, H), lambda i: (0, 0)),
        pl.BlockSpec((H, H), lambda i: (0, 0)),
        pl.BlockSpec((1, H), lambda i: (0, 0)),
        pl.BlockSpec((H, H), lambda i: (0, 0)),
    ],
    out_specs=[
        pl.BlockSpec((TN, H), lambda i: (i, 0)),
        pl.BlockSpec((TN, H), lambda i: (i, 0)),
    ],
    out_shape=[
        jax.ShapeDtypeStruct((N, H), jnp.float32),
        jax.ShapeDtypeStruct((N, H), jnp.float32),
    ],
)


def _final_body(agga_ref, aggb_ref, h_ref, batch_ref, l2w_ref, l2b_ref, lw_ref,
                lb_ref, fl1w_ref, fl1b_ref, fl2w_ref, fl2b_ref, pw_ref, pb_ref,
                out_ref):
    i = pl.program_id(0)
    agg = agga_ref[...] + aggb_ref[...]
    x = jnp.dot(agg, l2w_ref[...], preferred_element_type=jnp.float32)
    x = _ssp(x + l2b_ref[...])
    x = jnp.dot(x, lw_ref[...], preferred_element_type=jnp.float32) + lb_ref[...]
    h2 = h_ref[...] + x
    hf = _ssp(jnp.dot(h2, fl1w_ref[...], preferred_element_type=jnp.float32)
              + fl1b_ref[...])
    hf = jnp.dot(hf, fl2w_ref[...], preferred_element_type=jnp.float32)
    hf = hf + fl2b_ref[...]
    hp = jnp.dot(hf, pw_ref[...], preferred_element_type=jnp.float32)
    oh = (batch_ref[...] == lax.broadcasted_iota(jnp.int32, (TN, NGRAPH), 1))
    part = lax.dot_general(oh.astype(jnp.float32), hp,
                           (((0,), (0,)), ((), ())),
                           preferred_element_type=jnp.float32)

    @pl.when(i == 0)
    def _():
        out_ref[...] = part + pb_ref[...]

    @pl.when(i > 0)
    def _():
        out_ref[...] = out_ref[...] + part


_final = pl.pallas_call(
    _final_body,
    grid=(GN,),
    in_specs=[
        pl.BlockSpec((TN, H), lambda i: (i, 0)),
        pl.BlockSpec((TN, H), lambda i: (i, 0)),
        pl.BlockSpec((TN, H), lambda i: (i, 0)),
        pl.BlockSpec((TN, 1), lambda i: (i, 0)),
        pl.BlockSpec((H, H), lambda i: (0, 0)),
        pl.BlockSpec((1, H), lambda i: (0, 0)),
        pl.BlockSpec((H, H), lambda i: (0, 0)),
        pl.BlockSpec((1, H), lambda i: (0, 0)),
        pl.BlockSpec((H, HH), lambda i: (0, 0)),
        pl.BlockSpec((1, HH), lambda i: (0, 0)),
        pl.BlockSpec((HH, H), lambda i: (0, 0)),
        pl.BlockSpec((1, H), lambda i: (0, 0)),
        pl.BlockSpec((H, 1), lambda i: (0, 0)),
        pl.BlockSpec((1, 1), lambda i: (0, 0)),
    ],
    out_specs=pl.BlockSpec((NGRAPH, 1), lambda i: (0, 0)),
    out_shape=jax.ShapeDtypeStruct((NGRAPH, 1), jnp.float32),
)


def kernel(z, pos, batch, edge_index, emb, mw1_0, mb1_0, mw2_0, mb2_0, l1w_0,
           l2w_0, l2b_0, lw_0, lb_0, mw1_1, mb1_1, mw2_1, mb2_1, l1w_1, l2w_1,
           l2b_1, lw_1, lb_1, fl1w, fl1b, fl2w, fl2b, pw, pb):
    src = edge_index[0]
    dst = edge_index[1]
    srcp = jnp.pad(src, (0, 2 * K))
    dstp = jnp.pad(dst, (0, 2 * K))
    posx = pos[:, 0]
    posy = pos[:, 1]
    posz = pos[:, 2]
    z2 = z.reshape(N, 1).astype(jnp.int32)
    batch2 = batch.reshape(N, 1).astype(jnp.int32)

    d2 = _dist2(posx, posy, posz, src, dst)
    d2r = jnp.pad(d2, (0, EP - E)).reshape(ER, 128)
    wf0 = _filter(d2r, mw1_0, mb1_0.reshape(1, H), mw2_0, mb2_0.reshape(1, H))
    wf1 = _filter(d2r, mw1_1, mb1_1.reshape(1, H), mw2_1, mb2_1.reshape(1, H))
    h0, xl0 = _embed(z2, emb, l1w_0)
    agg0a, agg0b = _agg(xl0, wf0, srcp, dstp)
    h1, xl1 = _update(agg0a, agg0b, h0, l2w_0, l2b_0.reshape(1, H), lw_0,
                      lb_0.reshape(1, H), l1w_1)
    agg1a, agg1b = _agg(xl1, wf1, srcp, dstp)
    out = _final(agg1a, agg1b, h1, batch2, l2w_1, l2b_1.reshape(1, H),
                 lw_1, lb_1.reshape(1, H), fl1w, fl1b.reshape(1, HH), fl2w,
                 fl2b.reshape(1, H), pw, pb.reshape(1, 1))
    return out
